# baseline (speedup 1.0000x reference)
import functools

import jax
import jax.numpy as jnp
from jax import lax
from jax.experimental import pallas as pl
from jax.experimental.pallas import tpu as pltpu
from jax.experimental.pallas import tpu_sc as plsc

N = 10000
E = 320000
D = 128
R = 130
EPS = 1e-5

NC = 2
NS = 16
NW = NC * NS
CB = 128
EW = 10112
E_PAD = EW * NW
NCHUNK = EW // CB
CB2 = 64
N_PAD = 10240
ROWS_PER_TILE = N_PAD // NS
TRASH = N


_mesh = plsc.VectorSubcoreMesh(core_axis_name="c", subcore_axis_name="s")


def _worker_id():
    return lax.axis_index("c") * NS + lax.axis_index("s")


@functools.partial(
    pl.kernel,
    out_type=[
        jax.ShapeDtypeStruct((E_PAD,), jnp.float32),
        jax.ShapeDtypeStruct((E_PAD, D), jnp.float32),
        jax.ShapeDtypeStruct((NC, N_PAD, 16), jnp.float32),
    ],
    mesh=_mesh,
    compiler_params=pltpu.CompilerParams(needs_layout_passes=False,
                                         use_tc_tiling_on_sc=False),
    scratch_types=[
        pltpu.VMEM((R, D), jnp.float32),
        pltpu.VMEM((2, CB), jnp.int32),
        pltpu.VMEM((2, CB), jnp.int32),
        pltpu.VMEM((NCHUNK, CB), jnp.int32),
        pltpu.VMEM((1, CB), jnp.int32),
        pltpu.VMEM((2 * CB, D), jnp.float32),
        pltpu.VMEM((2 * CB, D), jnp.float32),
        pltpu.VMEM((CB, 16), jnp.float32),
        pltpu.VMEM((CB,), jnp.float32),
        pltpu.VMEM_SHARED((N_PAD, 16), jnp.float32),
        pltpu.SemaphoreType.DMA,
    ],
)
def _pass1(ent_hbm, rel_hbm, src_hbm, dstg_hbm, dsts3_hbm, relid_hbm,
           score_hbm, comp_hbm, stats_hbm,
           rel_v, sidx, dgidx, dsall, ridx, srows, drows, statrows,
           scorebuf, stats_acc, sem):
    cid = lax.axis_index("c")
    sid = lax.axis_index("s")
    wid = _worker_id()
    lane = lax.iota(jnp.int32, 16)
    zero16 = jnp.zeros((16,), jnp.float32)
    zero16i = jnp.zeros((16,), jnp.int32)
    onehot1 = jnp.where(lane == 1, 1.0, 0.0).astype(jnp.float32)

    pltpu.sync_copy(rel_hbm, rel_v)
    pltpu.sync_copy(dsts3_hbm.at[wid], dsall)

    def _zrow(i, _):
        statrows[i] = zero16
        return 0
    lax.fori_loop(0, CB, _zrow, 0)
    for k in range(ROWS_PER_TILE // CB):
        pltpu.sync_copy(
            statrows, stats_acc.at[pl.ds(sid * ROWS_PER_TILE + k * CB, CB)])

    def _irow(i, _):
        statrows[i] = onehot1
        return 0
    lax.fori_loop(0, CB, _irow, 0)
    plsc.subcore_barrier()

    def _start(c, b):
        base = wid * EW + c * CB
        pltpu.sync_copy(src_hbm.at[pl.ds(base, CB)], sidx.at[b])
        pltpu.sync_copy(dstg_hbm.at[pl.ds(base, CB)], dgidx.at[b])
        rb = b * CB
        pltpu.async_copy(ent_hbm.at[sidx.at[b]],
                         srows.at[pl.ds(rb, CB)], sem)
        pltpu.async_copy(ent_hbm.at[dgidx.at[b]],
                         drows.at[pl.ds(rb, CB)], sem)

    _start(0, 0)

    def _chunk(c, _):
        b = lax.rem(c, 2)
        rb = b * CB
        base = wid * EW + c * CB
        pltpu.make_async_copy(ent_hbm.at[sidx.at[b]],
                              srows.at[pl.ds(rb, CB)], sem).wait()
        pltpu.make_async_copy(ent_hbm.at[dgidx.at[b]],
                              drows.at[pl.ds(rb, CB)], sem).wait()

        @pl.when(c + 1 < NCHUNK)
        def _():
            _start(c + 1, 1 - b)

        pltpu.sync_copy(relid_hbm.at[pl.ds(base, CB)], ridx.at[0])

        def _group(v, _):
            sl = pl.ds(v * 16, 16)
            rid_vec = ridx[0, sl]
            evec = rb + v * 16 + lane
            evec0 = v * 16 + lane
            acc = zero16
            for j in range(D):
                jc = jnp.full((16,), j, jnp.int32)
                sc = plsc.load_gather(srows, [evec, jc])
                dc = plsc.load_gather(drows, [evec, jc])
                rc = plsc.load_gather(rel_v, [rid_vec, jc])
                cj = sc * rc
                acc = acc + cj * dc
                plsc.store_scatter(srows, [evec, jc], cj)
            scorebuf[sl] = acc
            plsc.store_scatter(statrows, [evec0, zero16i], acc)
            return 0
        lax.fori_loop(0, CB // 16, _group, 0)

        pltpu.sync_copy(scorebuf, score_hbm.at[pl.ds(base, CB)])
        pltpu.sync_copy(srows.at[pl.ds(rb, CB)],
                        comp_hbm.at[pl.ds(base, CB)])
        pltpu.sync_copy(statrows, stats_acc.at[dsall.at[c]], add=True)
        return 0
    lax.fori_loop(0, NCHUNK, _chunk, 0)

    plsc.subcore_barrier()
    row0 = sid * ROWS_PER_TILE
    pltpu.sync_copy(stats_acc.at[pl.ds(row0, ROWS_PER_TILE)],
                    stats_hbm.at[cid].at[pl.ds(row0, ROWS_PER_TILE)])


def _mean_body(stats_ref, m_ref):
    s = stats_ref[0] + stats_ref[1]
    m = s[:, 0:1] / jnp.maximum(s[:, 1:2], 1.0)
    m_ref[...] = jnp.reshape(m, (N_PAD // 128, 128))


def _seg_mean(stats):
    return pl.pallas_call(
        _mean_body,
        out_shape=jax.ShapeDtypeStruct((N_PAD // 128, 128), jnp.float32),
    )(stats)


@functools.partial(
    pl.kernel,
    out_type=[
        jax.ShapeDtypeStruct((NC, N_PAD, D), jnp.float32),
        jax.ShapeDtypeStruct((NC, N_PAD, 16), jnp.float32),
    ],
    mesh=_mesh,
    compiler_params=pltpu.CompilerParams(needs_layout_passes=False,
                                         use_tc_tiling_on_sc=False),
    scratch_types=[
        pltpu.VMEM((1, CB2), jnp.int32),
        pltpu.VMEM((N_PAD,), jnp.float32),
        pltpu.VMEM((CB2,), jnp.float32),
        pltpu.VMEM((CB2, D), jnp.float32),
        pltpu.VMEM((CB2, 16), jnp.float32),
        pltpu.VMEM_SHARED((N_PAD, D), jnp.float32),
        pltpu.VMEM_SHARED((N_PAD, 16), jnp.float32),
        pltpu.SemaphoreType.DMA,
    ],
)
def _pass2(comp_hbm, dsts_hbm, score_hbm, m_hbm,
           accn_hbm, accd_hbm,
           dsidx, m_v, scorebuf, scatbuf, exrows,
           acc, accd, sem):
    cid = lax.axis_index("c")
    sid = lax.axis_index("s")
    wid = _worker_id()
    lane = lax.iota(jnp.int32, 16)
    zero16 = jnp.zeros((16,), jnp.float32)

    pltpu.sync_copy(m_hbm, m_v)

    def _zrow(i, _):
        for j in range(D // 16):
            scatbuf[i, pl.ds(j * 16, 16)] = zero16
        exrows[i] = zero16
        return 0
    lax.fori_loop(0, CB2, _zrow, 0)
    for k in range(ROWS_PER_TILE // CB2):
        pltpu.sync_copy(scatbuf,
                        acc.at[pl.ds(sid * ROWS_PER_TILE + k * CB2, CB2)])
        pltpu.sync_copy(exrows,
                        accd.at[pl.ds(sid * ROWS_PER_TILE + k * CB2, CB2)])
    plsc.subcore_barrier()

    def _chunk(c, _):
        base = wid * EW + c * CB2
        pltpu.sync_copy(dsts_hbm.at[pl.ds(base, CB2)], dsidx.at[0])
        pltpu.sync_copy(score_hbm.at[pl.ds(base, CB2)], scorebuf)
        pltpu.sync_copy(comp_hbm.at[pl.ds(base, CB2)], scatbuf)

        def _group(v, _):
            sl = pl.ds(v * 16, 16)
            dstvec = dsidx[0, sl]
            mvec = plsc.load_gather(m_v, [dstvec])
            ex = jnp.exp(scorebuf[sl] - mvec)
            for l in range(16):
                e = v * 16 + l
                exl = ex[l]
                for j in range(D // 16):
                    slj = pl.ds(j * 16, 16)
                    scatbuf[e, slj] = scatbuf[e, slj] * exl
                exrows[e] = jnp.where(lane == 0, exl, 0.0)
            return 0
        lax.fori_loop(0, CB2 // 16, _group, 0)

        pltpu.sync_copy(scatbuf, acc.at[dsidx.at[0]], add=True)
        pltpu.sync_copy(exrows, accd.at[dsidx.at[0]], add=True)
        return 0
    lax.fori_loop(0, EW // CB2, _chunk, 0)

    plsc.subcore_barrier()
    row0 = sid * ROWS_PER_TILE
    pltpu.sync_copy(acc.at[pl.ds(row0, ROWS_PER_TILE)],
                    accn_hbm.at[cid].at[pl.ds(row0, ROWS_PER_TILE)])
    pltpu.sync_copy(accd.at[pl.ds(row0, ROWS_PER_TILE)],
                    accd_hbm.at[cid].at[pl.ds(row0, ROWS_PER_TILE)])


def _final_body(accn_ref, accd_ref, w_ref, g_ref, b_ref, out_ref):
    num = (accn_ref[0] + accn_ref[1])[0:N]
    den = (accd_ref[0] + accd_ref[1])[0:N, 0:1]
    neigh = num / jnp.maximum(den, 1e-30)
    out = jnp.dot(neigh, w_ref[...], preferred_element_type=jnp.float32)
    mean = jnp.mean(out, axis=0, keepdims=True)
    var = jnp.mean((out - mean) ** 2, axis=0, keepdims=True)
    out = (out - mean) / jnp.sqrt(var + EPS) * g_ref[...] + b_ref[...]
    out_ref[...] = jnp.tanh(out)


def _final(accn, accd, neigh_w, bn_gamma, bn_beta):
    return pl.pallas_call(
        _final_body,
        out_shape=jax.ShapeDtypeStruct((N, D), jnp.float32),
    )(accn, accd, neigh_w, bn_gamma.reshape(1, D), bn_beta.reshape(1, D))


def kernel(ent_emb, rel_emb, edge_index, rel_id, neigh_w, bn_gamma, bn_beta):
    src = edge_index[0]
    dst = edge_index[1]
    pad = E_PAD - E
    zpad = jnp.zeros((pad,), jnp.int32)
    src_p = jnp.concatenate([src, zpad])
    dstg_p = jnp.concatenate([dst, zpad])
    dsts_p = jnp.concatenate([dst, jnp.full((pad,), TRASH, jnp.int32)])
    rel_p = jnp.concatenate([rel_id, zpad])

    dsts3 = dsts_p.reshape(NW, NCHUNK, CB)
    score, comp, stats = _pass1(ent_emb, rel_emb, src_p, dstg_p, dsts3,
                                rel_p)
    m = _seg_mean(stats).reshape(N_PAD)
    accn, accd = _pass2(comp, dsts_p, score, m)
    return _final(accn, accd, neigh_w, bn_gamma, bn_beta)


# speedup vs baseline: 2.7388x; 2.7388x over previous
import functools

import jax
import jax.numpy as jnp
from jax import lax
from jax.experimental import pallas as pl
from jax.experimental.pallas import tpu as pltpu
from jax.experimental.pallas import tpu_sc as plsc

N = 10000
E = 320000
D = 128
R = 130
EPS = 1e-5

NC = 2
NS = 16
NW = NC * NS
CB = 128
EW = 10112
E_PAD = EW * NW
NCHUNK = EW // CB
CB2 = 64
N_PAD = 10240
ROWS_PER_TILE = N_PAD // NS
TRASH = N


_mesh = plsc.VectorSubcoreMesh(core_axis_name="c", subcore_axis_name="s")


def _worker_id():
    return lax.axis_index("c") * NS + lax.axis_index("s")


@functools.partial(
    pl.kernel,
    out_type=[
        jax.ShapeDtypeStruct((E_PAD,), jnp.float32),
        jax.ShapeDtypeStruct((E_PAD, D), jnp.float32),
        jax.ShapeDtypeStruct((NC, N_PAD, 16), jnp.float32),
    ],
    mesh=_mesh,
    compiler_params=pltpu.CompilerParams(needs_layout_passes=False,
                                         use_tc_tiling_on_sc=False),
    scratch_types=[
        pltpu.VMEM((R, D), jnp.float32),
        pltpu.VMEM((2, CB), jnp.int32),
        pltpu.VMEM((2, CB), jnp.int32),
        pltpu.VMEM((NCHUNK, CB), jnp.int32),
        pltpu.VMEM((1, CB), jnp.int32),
        pltpu.VMEM((2 * CB, D), jnp.float32),
        pltpu.VMEM((2 * CB, D), jnp.float32),
        pltpu.VMEM((CB, 16), jnp.float32),
        pltpu.VMEM((CB,), jnp.float32),
        pltpu.VMEM_SHARED((N_PAD, 16), jnp.float32),
        pltpu.SemaphoreType.DMA,
    ],
)
def _pass1(ent_hbm, rel_hbm, src_hbm, dstg_hbm, dsts3_hbm, relid_hbm,
           score_hbm, comp_hbm, stats_hbm,
           rel_v, sidx, dgidx, dsall, ridx, srows, drows, statrows,
           scorebuf, stats_acc, sem):
    cid = lax.axis_index("c")
    sid = lax.axis_index("s")
    wid = _worker_id()
    lane = lax.iota(jnp.int32, 16)
    zero16 = jnp.zeros((16,), jnp.float32)
    zero16i = jnp.zeros((16,), jnp.int32)
    onehot1 = jnp.where(lane == 1, 1.0, 0.0).astype(jnp.float32)

    pltpu.sync_copy(rel_hbm, rel_v)
    pltpu.sync_copy(dsts3_hbm.at[wid], dsall)

    def _zrow(i, _):
        statrows[i] = zero16
        return 0
    lax.fori_loop(0, CB, _zrow, 0)
    for k in range(ROWS_PER_TILE // CB):
        pltpu.sync_copy(
            statrows, stats_acc.at[pl.ds(sid * ROWS_PER_TILE + k * CB, CB)])

    def _irow(i, _):
        statrows[i] = onehot1
        return 0
    lax.fori_loop(0, CB, _irow, 0)
    plsc.subcore_barrier()

    def _start(c, b):
        base = wid * EW + c * CB
        pltpu.sync_copy(src_hbm.at[pl.ds(base, CB)], sidx.at[b])
        pltpu.sync_copy(dstg_hbm.at[pl.ds(base, CB)], dgidx.at[b])
        rb = b * CB
        pltpu.async_copy(ent_hbm.at[sidx.at[b]],
                         srows.at[pl.ds(rb, CB)], sem)
        pltpu.async_copy(ent_hbm.at[dgidx.at[b]],
                         drows.at[pl.ds(rb, CB)], sem)

    _start(0, 0)

    def _chunk(c, _):
        b = lax.rem(c, 2)
        rb = b * CB
        base = wid * EW + c * CB
        pltpu.make_async_copy(ent_hbm.at[sidx.at[b]],
                              srows.at[pl.ds(rb, CB)], sem).wait()
        pltpu.make_async_copy(ent_hbm.at[dgidx.at[b]],
                              drows.at[pl.ds(rb, CB)], sem).wait()

        @pl.when(c + 1 < NCHUNK)
        def _():
            _start(c + 1, 1 - b)

        pltpu.sync_copy(relid_hbm.at[pl.ds(base, CB)], ridx.at[0])

        def _group(v, _):
            sl = pl.ds(v * 16, 16)
            rid_vec = ridx[0, sl]
            svec = zero16
            for l in range(16):
                e = rb + v * 16 + l
                rid = rid_vec[l]
                acc = zero16
                for j in range(D // 16):
                    slj = pl.ds(j * 16, 16)
                    cj = srows[e, slj] * rel_v[rid, slj]
                    acc = acc + cj * drows[e, slj]
                    srows[e, slj] = cj
                s = jnp.sum(acc)
                svec = jnp.where(lane == l, s, svec)
                statrows[v * 16 + l] = jnp.where(lane == 0, s, onehot1)
            scorebuf[sl] = svec
            return 0
        lax.fori_loop(0, CB // 16, _group, 0)

        pltpu.sync_copy(scorebuf, score_hbm.at[pl.ds(base, CB)])
        pltpu.sync_copy(srows.at[pl.ds(rb, CB)],
                        comp_hbm.at[pl.ds(base, CB)])
        pltpu.sync_copy(statrows, stats_acc.at[dsall.at[c]], add=True)
        return 0
    lax.fori_loop(0, NCHUNK, _chunk, 0)

    plsc.subcore_barrier()
    row0 = sid * ROWS_PER_TILE
    pltpu.sync_copy(stats_acc.at[pl.ds(row0, ROWS_PER_TILE)],
                    stats_hbm.at[cid].at[pl.ds(row0, ROWS_PER_TILE)])


def _mean_body(stats_ref, m_ref):
    s = stats_ref[0] + stats_ref[1]
    m = s[:, 0:1] / jnp.maximum(s[:, 1:2], 1.0)
    m_ref[...] = jnp.reshape(m, (N_PAD // 128, 128))


def _seg_mean(stats):
    return pl.pallas_call(
        _mean_body,
        out_shape=jax.ShapeDtypeStruct((N_PAD // 128, 128), jnp.float32),
    )(stats)


@functools.partial(
    pl.kernel,
    out_type=[
        jax.ShapeDtypeStruct((NC, N_PAD, D), jnp.float32),
        jax.ShapeDtypeStruct((NC, N_PAD, 16), jnp.float32),
    ],
    mesh=_mesh,
    compiler_params=pltpu.CompilerParams(needs_layout_passes=False,
                                         use_tc_tiling_on_sc=False),
    scratch_types=[
        pltpu.VMEM((1, CB2), jnp.int32),
        pltpu.VMEM((N_PAD,), jnp.float32),
        pltpu.VMEM((CB2,), jnp.float32),
        pltpu.VMEM((CB2, D), jnp.float32),
        pltpu.VMEM((CB2, 16), jnp.float32),
        pltpu.VMEM_SHARED((N_PAD, D), jnp.float32),
        pltpu.VMEM_SHARED((N_PAD, 16), jnp.float32),
        pltpu.SemaphoreType.DMA,
    ],
)
def _pass2(comp_hbm, dsts_hbm, score_hbm, m_hbm,
           accn_hbm, accd_hbm,
           dsidx, m_v, scorebuf, scatbuf, exrows,
           acc, accd, sem):
    cid = lax.axis_index("c")
    sid = lax.axis_index("s")
    wid = _worker_id()
    lane = lax.iota(jnp.int32, 16)
    zero16 = jnp.zeros((16,), jnp.float32)

    pltpu.sync_copy(m_hbm, m_v)

    def _zrow(i, _):
        for j in range(D // 16):
            scatbuf[i, pl.ds(j * 16, 16)] = zero16
        exrows[i] = zero16
        return 0
    lax.fori_loop(0, CB2, _zrow, 0)
    for k in range(ROWS_PER_TILE // CB2):
        pltpu.sync_copy(scatbuf,
                        acc.at[pl.ds(sid * ROWS_PER_TILE + k * CB2, CB2)])
        pltpu.sync_copy(exrows,
                        accd.at[pl.ds(sid * ROWS_PER_TILE + k * CB2, CB2)])
    plsc.subcore_barrier()

    def _chunk(c, _):
        base = wid * EW + c * CB2
        pltpu.sync_copy(dsts_hbm.at[pl.ds(base, CB2)], dsidx.at[0])
        pltpu.sync_copy(score_hbm.at[pl.ds(base, CB2)], scorebuf)
        pltpu.sync_copy(comp_hbm.at[pl.ds(base, CB2)], scatbuf)

        def _group(v, _):
            sl = pl.ds(v * 16, 16)
            dstvec = dsidx[0, sl]
            mvec = plsc.load_gather(m_v, [dstvec])
            ex = jnp.exp(scorebuf[sl] - mvec)
            for l in range(16):
                e = v * 16 + l
                exl = ex[l]
                for j in range(D // 16):
                    slj = pl.ds(j * 16, 16)
                    scatbuf[e, slj] = scatbuf[e, slj] * exl
                exrows[e] = jnp.where(lane == 0, exl, 0.0)
            return 0
        lax.fori_loop(0, CB2 // 16, _group, 0)

        pltpu.sync_copy(scatbuf, acc.at[dsidx.at[0]], add=True)
        pltpu.sync_copy(exrows, accd.at[dsidx.at[0]], add=True)
        return 0
    lax.fori_loop(0, EW // CB2, _chunk, 0)

    plsc.subcore_barrier()
    row0 = sid * ROWS_PER_TILE
    pltpu.sync_copy(acc.at[pl.ds(row0, ROWS_PER_TILE)],
                    accn_hbm.at[cid].at[pl.ds(row0, ROWS_PER_TILE)])
    pltpu.sync_copy(accd.at[pl.ds(row0, ROWS_PER_TILE)],
                    accd_hbm.at[cid].at[pl.ds(row0, ROWS_PER_TILE)])


def _final_body(accn_ref, accd_ref, w_ref, g_ref, b_ref, out_ref):
    num = (accn_ref[0] + accn_ref[1])[0:N]
    den = (accd_ref[0] + accd_ref[1])[0:N, 0:1]
    neigh = num / jnp.maximum(den, 1e-30)
    out = jnp.dot(neigh, w_ref[...], preferred_element_type=jnp.float32)
    mean = jnp.mean(out, axis=0, keepdims=True)
    var = jnp.mean((out - mean) ** 2, axis=0, keepdims=True)
    out = (out - mean) / jnp.sqrt(var + EPS) * g_ref[...] + b_ref[...]
    out_ref[...] = jnp.tanh(out)


def _final(accn, accd, neigh_w, bn_gamma, bn_beta):
    return pl.pallas_call(
        _final_body,
        out_shape=jax.ShapeDtypeStruct((N, D), jnp.float32),
    )(accn, accd, neigh_w, bn_gamma.reshape(1, D), bn_beta.reshape(1, D))


def kernel(ent_emb, rel_emb, edge_index, rel_id, neigh_w, bn_gamma, bn_beta):
    src = edge_index[0]
    dst = edge_index[1]
    pad = E_PAD - E
    zpad = jnp.zeros((pad,), jnp.int32)
    src_p = jnp.concatenate([src, zpad])
    dstg_p = jnp.concatenate([dst, zpad])
    dsts_p = jnp.concatenate([dst, jnp.full((pad,), TRASH, jnp.int32)])
    rel_p = jnp.concatenate([rel_id, zpad])

    dsts3 = dsts_p.reshape(NW, NCHUNK, CB)
    score, comp, stats = _pass1(ent_emb, rel_emb, src_p, dstg_p, dsts3,
                                rel_p)
    m = _seg_mean(stats).reshape(N_PAD)
    accn, accd = _pass2(comp, dsts_p, score, m)
    return _final(accn, accd, neigh_w, bn_gamma, bn_beta)


# speedup vs baseline: 3.3593x; 1.2265x over previous
import functools

import jax
import jax.numpy as jnp
from jax import lax
from jax.experimental import pallas as pl
from jax.experimental.pallas import tpu as pltpu
from jax.experimental.pallas import tpu_sc as plsc

N = 10000
E = 320000
D = 128
R = 130
EPS = 1e-5

NC = 2
NS = 16
NW = NC * NS
CB = 128
EW = 10112
E_PAD = EW * NW
NCHUNK = EW // CB
CB2 = 64
N_PAD = 10240
ROWS_PER_TILE = N_PAD // NS
TRASH = N


_mesh = plsc.VectorSubcoreMesh(core_axis_name="c", subcore_axis_name="s")


def _worker_id():
    return lax.axis_index("c") * NS + lax.axis_index("s")


@functools.partial(
    pl.kernel,
    out_type=[
        jax.ShapeDtypeStruct((E_PAD,), jnp.float32),
        jax.ShapeDtypeStruct((E_PAD, D), jnp.float32),
        jax.ShapeDtypeStruct((NC, N_PAD, 16), jnp.float32),
    ],
    mesh=_mesh,
    compiler_params=pltpu.CompilerParams(needs_layout_passes=False,
                                         use_tc_tiling_on_sc=False),
    scratch_types=[
        pltpu.VMEM((R, D), jnp.float32),
        pltpu.VMEM((2, CB), jnp.int32),
        pltpu.VMEM((2, CB), jnp.int32),
        pltpu.VMEM((NCHUNK, CB), jnp.int32),
        pltpu.VMEM((1, CB), jnp.int32),
        pltpu.VMEM((2 * CB, D), jnp.float32),
        pltpu.VMEM((2 * CB, D), jnp.float32),
        pltpu.VMEM((CB, 16), jnp.float32),
        pltpu.VMEM((CB,), jnp.float32),
        pltpu.VMEM_SHARED((N_PAD, 16), jnp.float32),
        pltpu.SemaphoreType.DMA,
    ],
)
def _pass1(ent_hbm, rel_hbm, src_hbm, dstg_hbm, dsts3_hbm, relid_hbm,
           score_hbm, comp_hbm, stats_hbm,
           rel_v, sidx, dgidx, dsall, ridx, srows, drows, statrows,
           scorebuf, stats_acc, sem):
    cid = lax.axis_index("c")
    sid = lax.axis_index("s")
    wid = _worker_id()
    lane = lax.iota(jnp.int32, 16)
    zero16 = jnp.zeros((16,), jnp.float32)
    zero16i = jnp.zeros((16,), jnp.int32)
    onehot1 = jnp.where(lane == 1, 1.0, 0.0).astype(jnp.float32)

    pltpu.sync_copy(rel_hbm, rel_v)
    pltpu.sync_copy(dsts3_hbm.at[wid], dsall)

    def _zrow(i, _):
        statrows[i] = zero16
        return 0
    lax.fori_loop(0, CB, _zrow, 0)
    for k in range(ROWS_PER_TILE // CB):
        pltpu.sync_copy(
            statrows, stats_acc.at[pl.ds(sid * ROWS_PER_TILE + k * CB, CB)])

    def _irow(i, _):
        statrows[i] = onehot1
        return 0
    lax.fori_loop(0, CB, _irow, 0)
    plsc.subcore_barrier()

    def _start(c, b):
        base = wid * EW + c * CB
        pltpu.sync_copy(src_hbm.at[pl.ds(base, CB)], sidx.at[b])
        pltpu.sync_copy(dstg_hbm.at[pl.ds(base, CB)], dgidx.at[b])
        rb = b * CB
        pltpu.async_copy(ent_hbm.at[sidx.at[b]],
                         srows.at[pl.ds(rb, CB)], sem)
        pltpu.async_copy(ent_hbm.at[dgidx.at[b]],
                         drows.at[pl.ds(rb, CB)], sem)

    _start(0, 0)

    def _chunk(c, _):
        b = lax.rem(c, 2)
        rb = b * CB
        base = wid * EW + c * CB
        pltpu.make_async_copy(ent_hbm.at[sidx.at[b]],
                              srows.at[pl.ds(rb, CB)], sem).wait()
        pltpu.make_async_copy(ent_hbm.at[dgidx.at[b]],
                              drows.at[pl.ds(rb, CB)], sem).wait()

        @pl.when(c + 1 < NCHUNK)
        def _():
            _start(c + 1, 1 - b)

        pltpu.sync_copy(relid_hbm.at[pl.ds(base, CB)], ridx.at[0])

        def _group(v, _):
            sl = pl.ds(v * 16, 16)
            rid_vec = ridx[0, sl]
            svec = zero16
            for l in range(16):
                e = rb + v * 16 + l
                rid = rid_vec[l]
                acc = zero16
                for j in range(D // 16):
                    slj = pl.ds(j * 16, 16)
                    cj = srows[e, slj] * rel_v[rid, slj]
                    acc = acc + cj * drows[e, slj]
                    srows[e, slj] = cj
                s = jnp.sum(acc)
                svec = jnp.where(lane == l, s, svec)
                statrows[v * 16 + l] = jnp.where(lane == 0, s, onehot1)
            scorebuf[sl] = svec
            return 0
        lax.fori_loop(0, CB // 16, _group, 0)

        pltpu.sync_copy(scorebuf, score_hbm.at[pl.ds(base, CB)])
        pltpu.sync_copy(srows.at[pl.ds(rb, CB)],
                        comp_hbm.at[pl.ds(base, CB)])
        pltpu.sync_copy(statrows, stats_acc.at[dsall.at[c]], add=True)
        return 0
    lax.fori_loop(0, NCHUNK, _chunk, 0)

    plsc.subcore_barrier()
    row0 = sid * ROWS_PER_TILE
    pltpu.sync_copy(stats_acc.at[pl.ds(row0, ROWS_PER_TILE)],
                    stats_hbm.at[cid].at[pl.ds(row0, ROWS_PER_TILE)])


def _mean_body(stats_ref, m_ref):
    s = stats_ref[0] + stats_ref[1]
    m = s[:, 0:1] / jnp.maximum(s[:, 1:2], 1.0)
    m_ref[...] = jnp.reshape(m, (N_PAD // 128, 128))


def _seg_mean(stats):
    return pl.pallas_call(
        _mean_body,
        out_shape=jax.ShapeDtypeStruct((N_PAD // 128, 128), jnp.float32),
    )(stats)


NCH2 = EW // CB2


@functools.partial(
    pl.kernel,
    out_type=[
        jax.ShapeDtypeStruct((NC, N_PAD, D), jnp.float32),
        jax.ShapeDtypeStruct((NC, NS, N_PAD), jnp.float32),
    ],
    mesh=_mesh,
    compiler_params=pltpu.CompilerParams(needs_layout_passes=False,
                                         use_tc_tiling_on_sc=False),
    scratch_types=[
        pltpu.VMEM((3, CB2), jnp.int32),
        pltpu.VMEM((N_PAD,), jnp.float32),
        pltpu.VMEM((N_PAD,), jnp.float32),
        pltpu.VMEM((3, CB2), jnp.float32),
        pltpu.VMEM((3 * CB2, D), jnp.float32),
        pltpu.VMEM_SHARED((N_PAD, D), jnp.float32),
        pltpu.SemaphoreType.DMA,
        pltpu.SemaphoreType.DMA,
    ],
)
def _pass2(comp_hbm, dsts_hbm, score_hbm, m_hbm,
           accn_hbm, den_hbm,
           dsidx, m_v, den_v, scorebuf, scatbuf,
           acc, sem_l, sem_s):
    cid = lax.axis_index("c")
    sid = lax.axis_index("s")
    wid = _worker_id()
    lane = lax.iota(jnp.int32, 16)
    zero16 = jnp.zeros((16,), jnp.float32)

    pltpu.sync_copy(m_hbm, m_v)

    def _zden(i, _):
        den_v[pl.ds(i * 16, 16)] = zero16
        return 0
    lax.fori_loop(0, N_PAD // 16, _zden, 0)

    def _zrow(i, _):
        for j in range(D // 16):
            scatbuf[i, pl.ds(j * 16, 16)] = zero16
        return 0
    lax.fori_loop(0, CB2, _zrow, 0)
    for k in range(ROWS_PER_TILE // CB2):
        pltpu.sync_copy(scatbuf.at[pl.ds(0, CB2)],
                        acc.at[pl.ds(sid * ROWS_PER_TILE + k * CB2, CB2)])
    plsc.subcore_barrier()

    def _load(c):
        h = lax.rem(c, 3)
        base = wid * EW + c * CB2
        pltpu.sync_copy(dsts_hbm.at[pl.ds(base, CB2)], dsidx.at[h])
        pltpu.sync_copy(score_hbm.at[pl.ds(base, CB2)], scorebuf.at[h])
        pltpu.async_copy(comp_hbm.at[pl.ds(base, CB2)],
                         scatbuf.at[pl.ds(h * CB2, CB2)], sem_l)

    _load(0)
    _load(1)

    def _chunk(c, _):
        h = lax.rem(c, 3)
        hp = lax.rem(c + 2, 3)
        base = wid * EW + c * CB2
        rb = h * CB2
        pltpu.make_async_copy(comp_hbm.at[pl.ds(base, CB2)],
                              scatbuf.at[pl.ds(rb, CB2)], sem_l).wait()

        def _group(v, _):
            sl = pl.ds(v * 16, 16)
            dstvec = dsidx[h, sl]
            mvec = plsc.load_gather(m_v, [dstvec])
            ex = jnp.exp(scorebuf[h, sl] - mvec)
            plsc.addupdate_scatter(den_v, [dstvec], ex)
            for l in range(16):
                e = rb + v * 16 + l
                exl = ex[l]
                for j in range(D // 16):
                    slj = pl.ds(j * 16, 16)
                    scatbuf[e, slj] = scatbuf[e, slj] * exl
            return 0
        lax.fori_loop(0, CB2 // 16, _group, 0)

        @pl.when(c > 0)
        def _():
            pltpu.make_async_copy(
                scatbuf.at[pl.ds(hp * CB2, CB2)],
                acc.at[dsidx.at[hp]], sem_s).wait()

        pltpu.async_copy(scatbuf.at[pl.ds(rb, CB2)],
                         acc.at[dsidx.at[h]], sem_s, add=True)

        @pl.when(c + 2 < NCH2)
        def _():
            _load(c + 2)
        return 0
    lax.fori_loop(0, NCH2, _chunk, 0)

    hl = lax.rem(NCH2 - 1, 3)
    pltpu.make_async_copy(scatbuf.at[pl.ds(hl * CB2, CB2)],
                          acc.at[dsidx.at[hl]], sem_s).wait()

    plsc.subcore_barrier()
    row0 = sid * ROWS_PER_TILE
    pltpu.sync_copy(acc.at[pl.ds(row0, ROWS_PER_TILE)],
                    accn_hbm.at[cid].at[pl.ds(row0, ROWS_PER_TILE)])
    pltpu.sync_copy(den_v, den_hbm.at[cid].at[sid])


def _final_body(accn_ref, den_ref, w_ref, g_ref, b_ref, out_ref):
    num = (accn_ref[0] + accn_ref[1])[0:N]
    den = jnp.sum(den_ref[...], axis=(0, 1))[0:N, None]
    neigh = num / jnp.maximum(den, 1e-30)
    out = jnp.dot(neigh, w_ref[...], preferred_element_type=jnp.float32)
    mean = jnp.mean(out, axis=0, keepdims=True)
    var = jnp.mean((out - mean) ** 2, axis=0, keepdims=True)
    out = (out - mean) / jnp.sqrt(var + EPS) * g_ref[...] + b_ref[...]
    out_ref[...] = jnp.tanh(out)


def _final(accn, dens, neigh_w, bn_gamma, bn_beta):
    return pl.pallas_call(
        _final_body,
        out_shape=jax.ShapeDtypeStruct((N, D), jnp.float32),
    )(accn, dens, neigh_w, bn_gamma.reshape(1, D), bn_beta.reshape(1, D))


def kernel(ent_emb, rel_emb, edge_index, rel_id, neigh_w, bn_gamma, bn_beta):
    src = edge_index[0]
    dst = edge_index[1]
    pad = E_PAD - E
    zpad = jnp.zeros((pad,), jnp.int32)
    src_p = jnp.concatenate([src, zpad])
    dstg_p = jnp.concatenate([dst, zpad])
    dsts_p = jnp.concatenate([dst, jnp.full((pad,), TRASH, jnp.int32)])
    rel_p = jnp.concatenate([rel_id, zpad])

    dsts3 = dsts_p.reshape(NW, NCHUNK, CB)
    score, comp, stats = _pass1(ent_emb, rel_emb, src_p, dstg_p, dsts3,
                                rel_p)
    m = _seg_mean(stats).reshape(N_PAD)
    accn, dens = _pass2(comp, dsts_p, score, m)
    return _final(accn, dens, neigh_w, bn_gamma, bn_beta)


# speedup vs baseline: 3.9641x; 1.1801x over previous
import functools

import jax
import jax.numpy as jnp
from jax import lax
from jax.experimental import pallas as pl
from jax.experimental.pallas import tpu as pltpu
from jax.experimental.pallas import tpu_sc as plsc

N = 10000
E = 320000
D = 128
R = 130
EPS = 1e-5

NC = 2
NS = 16
NW = NC * NS
CB = 128
EW = 10112
E_PAD = EW * NW
NCHUNK = EW // CB
CB2 = 64
N_PAD = 10240
ROWS_PER_TILE = N_PAD // NS
TRASH = N


_mesh = plsc.VectorSubcoreMesh(core_axis_name="c", subcore_axis_name="s")


def _worker_id():
    return lax.axis_index("c") * NS + lax.axis_index("s")


@functools.partial(
    pl.kernel,
    out_type=[
        jax.ShapeDtypeStruct((E_PAD,), jnp.float32),
        jax.ShapeDtypeStruct((E_PAD, D), jnp.float32),
        jax.ShapeDtypeStruct((NC, NS, N_PAD), jnp.float32),
        jax.ShapeDtypeStruct((NC, NS, N_PAD), jnp.float32),
    ],
    mesh=_mesh,
    compiler_params=pltpu.CompilerParams(needs_layout_passes=False,
                                         use_tc_tiling_on_sc=False),
    scratch_types=[
        pltpu.VMEM((R, D), jnp.float32),
        pltpu.VMEM((NCHUNK, CB), jnp.int32),
        pltpu.VMEM((NCHUNK, CB), jnp.int32),
        pltpu.VMEM((2, CB), jnp.int32),
        pltpu.VMEM((2 * CB, D), jnp.float32),
        pltpu.VMEM((2 * CB, D), jnp.float32),
        pltpu.VMEM((CB,), jnp.float32),
        pltpu.VMEM((N_PAD,), jnp.float32),
        pltpu.VMEM((N_PAD,), jnp.float32),
        pltpu.SemaphoreType.DMA,
    ],
)
def _pass1(ent_hbm, rel_hbm, src3_hbm, dstg3_hbm, relid_hbm,
           score_hbm, comp_hbm, sums_hbm, cnts_hbm,
           rel_v, sall, dall, ridx, srows, drows, scorebuf,
           sum_v, cnt_v, sem):
    cid = lax.axis_index("c")
    sid = lax.axis_index("s")
    wid = _worker_id()
    lane = lax.iota(jnp.int32, 16)
    zero16 = jnp.zeros((16,), jnp.float32)
    ones16 = jnp.ones((16,), jnp.float32)

    pltpu.sync_copy(rel_hbm, rel_v)
    pltpu.sync_copy(src3_hbm.at[wid], sall)
    pltpu.sync_copy(dstg3_hbm.at[wid], dall)

    def _zden(i, _):
        sum_v[pl.ds(i * 16, 16)] = zero16
        cnt_v[pl.ds(i * 16, 16)] = zero16
        return 0
    lax.fori_loop(0, N_PAD // 16, _zden, 0)

    def _start(c, b):
        base = wid * EW + c * CB
        rb = b * CB
        pltpu.async_copy(relid_hbm.at[pl.ds(base, CB)], ridx.at[b], sem)
        pltpu.async_copy(ent_hbm.at[sall.at[c]],
                         srows.at[pl.ds(rb, CB)], sem)
        pltpu.async_copy(ent_hbm.at[dall.at[c]],
                         drows.at[pl.ds(rb, CB)], sem)

    _start(0, 0)

    def _chunk(c, _):
        b = lax.rem(c, 2)
        rb = b * CB
        base = wid * EW + c * CB
        pltpu.make_async_copy(relid_hbm.at[pl.ds(base, CB)],
                              ridx.at[b], sem).wait()
        pltpu.make_async_copy(ent_hbm.at[sall.at[c]],
                              srows.at[pl.ds(rb, CB)], sem).wait()
        pltpu.make_async_copy(ent_hbm.at[dall.at[c]],
                              drows.at[pl.ds(rb, CB)], sem).wait()

        @pl.when(c + 1 < NCHUNK)
        def _():
            _start(c + 1, 1 - b)

        def _group(v, _):
            sl = pl.ds(v * 16, 16)
            rid_vec = ridx[b, sl]
            svec = zero16
            for l in range(16):
                e = rb + v * 16 + l
                rid = rid_vec[l]
                acc = zero16
                for j in range(D // 16):
                    slj = pl.ds(j * 16, 16)
                    cj = srows[e, slj] * rel_v[rid, slj]
                    acc = acc + cj * drows[e, slj]
                    srows[e, slj] = cj
                s = jnp.sum(acc)
                svec = jnp.where(lane == l, s, svec)
            scorebuf[sl] = svec
            dstvec = dall[c, sl]
            plsc.addupdate_scatter(sum_v, [dstvec], svec)
            plsc.addupdate_scatter(cnt_v, [dstvec], ones16)
            return 0
        lax.fori_loop(0, CB // 16, _group, 0)

        pltpu.sync_copy(scorebuf, score_hbm.at[pl.ds(base, CB)])
        pltpu.sync_copy(srows.at[pl.ds(rb, CB)],
                        comp_hbm.at[pl.ds(base, CB)])
        return 0
    lax.fori_loop(0, NCHUNK, _chunk, 0)

    pltpu.sync_copy(sum_v, sums_hbm.at[cid].at[sid])
    pltpu.sync_copy(cnt_v, cnts_hbm.at[cid].at[sid])


def _mean_body(sums_ref, cnts_ref, m_ref):
    s = jnp.sum(sums_ref[...], axis=(0, 1))
    n = jnp.sum(cnts_ref[...], axis=(0, 1))
    m = s / jnp.maximum(n, 1.0)
    m_ref[...] = jnp.reshape(m, (N_PAD // 128, 128))


def _seg_mean(sums, cnts):
    return pl.pallas_call(
        _mean_body,
        out_shape=jax.ShapeDtypeStruct((N_PAD // 128, 128), jnp.float32),
    )(sums, cnts)


NCH2 = EW // CB2


@functools.partial(
    pl.kernel,
    out_type=[
        jax.ShapeDtypeStruct((NC, N_PAD, D), jnp.float32),
        jax.ShapeDtypeStruct((NC, NS, N_PAD), jnp.float32),
    ],
    mesh=_mesh,
    compiler_params=pltpu.CompilerParams(needs_layout_passes=False,
                                         use_tc_tiling_on_sc=False),
    scratch_types=[
        pltpu.VMEM((3, CB2), jnp.int32),
        pltpu.VMEM((N_PAD,), jnp.float32),
        pltpu.VMEM((N_PAD,), jnp.float32),
        pltpu.VMEM((3, CB2), jnp.float32),
        pltpu.VMEM((3 * CB2, D), jnp.float32),
        pltpu.VMEM_SHARED((N_PAD, D), jnp.float32),
        pltpu.SemaphoreType.DMA,
        pltpu.SemaphoreType.DMA,
    ],
)
def _pass2(comp_hbm, dsts_hbm, score_hbm, m_hbm,
           accn_hbm, den_hbm,
           dsidx, m_v, den_v, scorebuf, scatbuf,
           acc, sem_l, sem_s):
    cid = lax.axis_index("c")
    sid = lax.axis_index("s")
    wid = _worker_id()
    lane = lax.iota(jnp.int32, 16)
    zero16 = jnp.zeros((16,), jnp.float32)

    pltpu.sync_copy(m_hbm, m_v)

    def _zden(i, _):
        den_v[pl.ds(i * 16, 16)] = zero16
        return 0
    lax.fori_loop(0, N_PAD // 16, _zden, 0)

    def _zrow(i, _):
        for j in range(D // 16):
            scatbuf[i, pl.ds(j * 16, 16)] = zero16
        return 0
    lax.fori_loop(0, CB2, _zrow, 0)
    for k in range(ROWS_PER_TILE // CB2):
        pltpu.sync_copy(scatbuf.at[pl.ds(0, CB2)],
                        acc.at[pl.ds(sid * ROWS_PER_TILE + k * CB2, CB2)])
    plsc.subcore_barrier()

    def _load(c):
        h = lax.rem(c, 3)
        base = wid * EW + c * CB2
        pltpu.sync_copy(dsts_hbm.at[pl.ds(base, CB2)], dsidx.at[h])
        pltpu.sync_copy(score_hbm.at[pl.ds(base, CB2)], scorebuf.at[h])
        pltpu.async_copy(comp_hbm.at[pl.ds(base, CB2)],
                         scatbuf.at[pl.ds(h * CB2, CB2)], sem_l)

    _load(0)
    _load(1)

    def _chunk(c, _):
        h = lax.rem(c, 3)
        hp = lax.rem(c + 2, 3)
        base = wid * EW + c * CB2
        rb = h * CB2
        pltpu.make_async_copy(comp_hbm.at[pl.ds(base, CB2)],
                              scatbuf.at[pl.ds(rb, CB2)], sem_l).wait()

        def _group(v, _):
            sl = pl.ds(v * 16, 16)
            dstvec = dsidx[h, sl]
            mvec = plsc.load_gather(m_v, [dstvec])
            ex = jnp.exp(scorebuf[h, sl] - mvec)
            plsc.addupdate_scatter(den_v, [dstvec], ex)
            for l in range(16):
                e = rb + v * 16 + l
                exl = ex[l]
                for j in range(D // 16):
                    slj = pl.ds(j * 16, 16)
                    scatbuf[e, slj] = scatbuf[e, slj] * exl
            return 0
        lax.fori_loop(0, CB2 // 16, _group, 0)

        @pl.when(c > 0)
        def _():
            pltpu.make_async_copy(
                scatbuf.at[pl.ds(hp * CB2, CB2)],
                acc.at[dsidx.at[hp]], sem_s).wait()

        pltpu.async_copy(scatbuf.at[pl.ds(rb, CB2)],
                         acc.at[dsidx.at[h]], sem_s, add=True)

        @pl.when(c + 2 < NCH2)
        def _():
            _load(c + 2)
        return 0
    lax.fori_loop(0, NCH2, _chunk, 0)

    hl = lax.rem(NCH2 - 1, 3)
    pltpu.make_async_copy(scatbuf.at[pl.ds(hl * CB2, CB2)],
                          acc.at[dsidx.at[hl]], sem_s).wait()

    plsc.subcore_barrier()
    row0 = sid * ROWS_PER_TILE
    pltpu.sync_copy(acc.at[pl.ds(row0, ROWS_PER_TILE)],
                    accn_hbm.at[cid].at[pl.ds(row0, ROWS_PER_TILE)])
    pltpu.sync_copy(den_v, den_hbm.at[cid].at[sid])


def _final_body(accn_ref, den_ref, w_ref, g_ref, b_ref, out_ref):
    num = (accn_ref[0] + accn_ref[1])[0:N]
    den = jnp.sum(den_ref[...], axis=(0, 1))[0:N, None]
    neigh = num / jnp.maximum(den, 1e-30)
    out = jnp.dot(neigh, w_ref[...], preferred_element_type=jnp.float32)
    mean = jnp.mean(out, axis=0, keepdims=True)
    var = jnp.mean((out - mean) ** 2, axis=0, keepdims=True)
    out = (out - mean) / jnp.sqrt(var + EPS) * g_ref[...] + b_ref[...]
    out_ref[...] = jnp.tanh(out)


def _final(accn, dens, neigh_w, bn_gamma, bn_beta):
    return pl.pallas_call(
        _final_body,
        out_shape=jax.ShapeDtypeStruct((N, D), jnp.float32),
    )(accn, dens, neigh_w, bn_gamma.reshape(1, D), bn_beta.reshape(1, D))


def kernel(ent_emb, rel_emb, edge_index, rel_id, neigh_w, bn_gamma, bn_beta):
    src = edge_index[0]
    dst = edge_index[1]
    pad = E_PAD - E
    zpad = jnp.zeros((pad,), jnp.int32)
    src_p = jnp.concatenate([src, zpad])
    dstg_p = jnp.concatenate([dst, zpad])
    dsts_p = jnp.concatenate([dst, jnp.full((pad,), TRASH, jnp.int32)])
    rel_p = jnp.concatenate([rel_id, zpad])

    src3 = src_p.reshape(NW, NCHUNK, CB)
    dstg3 = dstg_p.reshape(NW, NCHUNK, CB)
    score, comp, sums, cnts = _pass1(ent_emb, rel_emb, src3, dstg3, rel_p)
    m = _seg_mean(sums, cnts).reshape(N_PAD)
    accn, dens = _pass2(comp, dsts_p, score, m)
    return _final(accn, dens, neigh_w, bn_gamma, bn_beta)


# speedup vs baseline: 4.3731x; 1.1032x over previous
import functools

import jax
import jax.numpy as jnp
from jax import lax
from jax.experimental import pallas as pl
from jax.experimental.pallas import tpu as pltpu
from jax.experimental.pallas import tpu_sc as plsc

N = 10000
E = 320000
D = 128
R = 130
EPS = 1e-5

NC = 2
NS = 16
NW = NC * NS
CB = 128
EW = 10112
E_PAD = EW * NW
NCHUNK = EW // CB
CB2 = 64
N_PAD = 10240
ROWS_PER_TILE = N_PAD // NS
TRASH = N


_mesh = plsc.VectorSubcoreMesh(core_axis_name="c", subcore_axis_name="s")


def _worker_id():
    return lax.axis_index("c") * NS + lax.axis_index("s")


@functools.partial(
    pl.kernel,
    out_type=[
        jax.ShapeDtypeStruct((E_PAD,), jnp.float32),
        jax.ShapeDtypeStruct((E_PAD, D), jnp.float32),
        jax.ShapeDtypeStruct((NC, NS, N_PAD), jnp.float32),
        jax.ShapeDtypeStruct((NC, NS, N_PAD), jnp.float32),
    ],
    mesh=_mesh,
    compiler_params=pltpu.CompilerParams(needs_layout_passes=False,
                                         use_tc_tiling_on_sc=False),
    scratch_types=[
        pltpu.VMEM((R, D), jnp.float32),
        pltpu.VMEM((NCHUNK, CB), jnp.int32),
        pltpu.VMEM((NCHUNK, CB), jnp.int32),
        pltpu.VMEM((2, CB), jnp.int32),
        pltpu.VMEM((2 * CB, D), jnp.float32),
        pltpu.VMEM((2 * CB, D), jnp.float32),
        pltpu.VMEM((2, CB), jnp.float32),
        pltpu.VMEM((N_PAD,), jnp.float32),
        pltpu.VMEM((N_PAD,), jnp.float32),
        pltpu.SemaphoreType.DMA,
        pltpu.SemaphoreType.DMA,
    ],
)
def _pass1(ent_hbm, rel_hbm, src3_hbm, dstg3_hbm, relid_hbm,
           score_hbm, comp_hbm, sums_hbm, cnts_hbm,
           rel_v, sall, dall, ridx, srows, drows, scorebuf,
           sum_v, cnt_v, sem, sem_w):
    cid = lax.axis_index("c")
    sid = lax.axis_index("s")
    wid = _worker_id()
    lane = lax.iota(jnp.int32, 16)
    zero16 = jnp.zeros((16,), jnp.float32)
    ones16 = jnp.ones((16,), jnp.float32)

    pltpu.sync_copy(rel_hbm, rel_v)
    pltpu.sync_copy(src3_hbm.at[wid], sall)
    pltpu.sync_copy(dstg3_hbm.at[wid], dall)

    def _zden(i, _):
        sum_v[pl.ds(i * 16, 16)] = zero16
        cnt_v[pl.ds(i * 16, 16)] = zero16
        return 0
    lax.fori_loop(0, N_PAD // 16, _zden, 0)

    def _wait_writes(cw, b):
        basew = wid * EW + cw * CB
        rbw = b * CB
        pltpu.make_async_copy(scorebuf.at[b],
                              score_hbm.at[pl.ds(basew, CB)], sem_w).wait()
        pltpu.make_async_copy(srows.at[pl.ds(rbw, CB)],
                              comp_hbm.at[pl.ds(basew, CB)], sem_w).wait()

    def _start(c, b):
        base = wid * EW + c * CB
        rb = b * CB

        @pl.when(c >= 2)
        def _():
            _wait_writes(c - 2, b)

        pltpu.async_copy(relid_hbm.at[pl.ds(base, CB)], ridx.at[b], sem)
        pltpu.async_copy(ent_hbm.at[sall.at[c]],
                         srows.at[pl.ds(rb, CB)], sem)
        pltpu.async_copy(ent_hbm.at[dall.at[c]],
                         drows.at[pl.ds(rb, CB)], sem)

    _start(0, 0)

    def _chunk(c, _):
        b = lax.rem(c, 2)
        rb = b * CB
        base = wid * EW + c * CB
        pltpu.make_async_copy(relid_hbm.at[pl.ds(base, CB)],
                              ridx.at[b], sem).wait()
        pltpu.make_async_copy(ent_hbm.at[sall.at[c]],
                              srows.at[pl.ds(rb, CB)], sem).wait()
        pltpu.make_async_copy(ent_hbm.at[dall.at[c]],
                              drows.at[pl.ds(rb, CB)], sem).wait()

        @pl.when(c + 1 < NCHUNK)
        def _():
            _start(c + 1, 1 - b)

        def _group(v, _):
            sl = pl.ds(v * 16, 16)
            rid_vec = ridx[b, sl]
            svec = zero16
            for l in range(16):
                e = rb + v * 16 + l
                rid = rid_vec[l]
                acc = zero16
                for j in range(D // 16):
                    slj = pl.ds(j * 16, 16)
                    cj = srows[e, slj] * rel_v[rid, slj]
                    acc = acc + cj * drows[e, slj]
                    srows[e, slj] = cj
                s = jnp.sum(acc)
                svec = jnp.where(lane == l, s, svec)
            scorebuf[b, sl] = svec
            dstvec = dall[c, sl]
            plsc.addupdate_scatter(sum_v, [dstvec], svec)
            plsc.addupdate_scatter(cnt_v, [dstvec], ones16)
            return 0
        lax.fori_loop(0, CB // 16, _group, 0)

        pltpu.async_copy(scorebuf.at[b], score_hbm.at[pl.ds(base, CB)],
                         sem_w)
        pltpu.async_copy(srows.at[pl.ds(rb, CB)],
                         comp_hbm.at[pl.ds(base, CB)], sem_w)
        return 0
    lax.fori_loop(0, NCHUNK, _chunk, 0)

    for cc in (NCHUNK - 2, NCHUNK - 1):
        _wait_writes(cc, cc % 2)

    pltpu.sync_copy(sum_v, sums_hbm.at[cid].at[sid])
    pltpu.sync_copy(cnt_v, cnts_hbm.at[cid].at[sid])


def _mean_body(sums_ref, cnts_ref, m_ref):
    s = jnp.sum(sums_ref[...], axis=(0, 1))
    n = jnp.sum(cnts_ref[...], axis=(0, 1))
    m = s / jnp.maximum(n, 1.0)
    m_ref[...] = jnp.reshape(m, (N_PAD // 128, 128))


def _seg_mean(sums, cnts):
    return pl.pallas_call(
        _mean_body,
        out_shape=jax.ShapeDtypeStruct((N_PAD // 128, 128), jnp.float32),
    )(sums, cnts)


NCH2 = EW // CB2


@functools.partial(
    pl.kernel,
    out_type=[
        jax.ShapeDtypeStruct((NC, N_PAD, D), jnp.float32),
        jax.ShapeDtypeStruct((NC, NS, N_PAD), jnp.float32),
    ],
    mesh=_mesh,
    compiler_params=pltpu.CompilerParams(needs_layout_passes=False,
                                         use_tc_tiling_on_sc=False),
    scratch_types=[
        pltpu.VMEM((3, CB2), jnp.int32),
        pltpu.VMEM((N_PAD,), jnp.float32),
        pltpu.VMEM((N_PAD,), jnp.float32),
        pltpu.VMEM((3, CB2), jnp.float32),
        pltpu.VMEM((3 * CB2, D), jnp.float32),
        pltpu.VMEM_SHARED((N_PAD, D), jnp.float32),
        pltpu.SemaphoreType.DMA,
        pltpu.SemaphoreType.DMA,
    ],
)
def _pass2(comp_hbm, dsts_hbm, score_hbm, m_hbm,
           accn_hbm, den_hbm,
           dsidx, m_v, den_v, scorebuf, scatbuf,
           acc, sem_l, sem_s):
    cid = lax.axis_index("c")
    sid = lax.axis_index("s")
    wid = _worker_id()
    lane = lax.iota(jnp.int32, 16)
    zero16 = jnp.zeros((16,), jnp.float32)

    pltpu.sync_copy(m_hbm, m_v)

    def _zden(i, _):
        den_v[pl.ds(i * 16, 16)] = zero16
        return 0
    lax.fori_loop(0, N_PAD // 16, _zden, 0)

    def _zrow(i, _):
        for j in range(D // 16):
            scatbuf[i, pl.ds(j * 16, 16)] = zero16
        return 0
    lax.fori_loop(0, CB2, _zrow, 0)
    for k in range(ROWS_PER_TILE // CB2):
        pltpu.sync_copy(scatbuf.at[pl.ds(0, CB2)],
                        acc.at[pl.ds(sid * ROWS_PER_TILE + k * CB2, CB2)])
    plsc.subcore_barrier()

    def _load(c):
        h = lax.rem(c, 3)
        base = wid * EW + c * CB2
        pltpu.async_copy(dsts_hbm.at[pl.ds(base, CB2)], dsidx.at[h], sem_l)
        pltpu.async_copy(score_hbm.at[pl.ds(base, CB2)], scorebuf.at[h],
                         sem_l)
        pltpu.async_copy(comp_hbm.at[pl.ds(base, CB2)],
                         scatbuf.at[pl.ds(h * CB2, CB2)], sem_l)

    _load(0)
    _load(1)

    def _chunk(c, _):
        h = lax.rem(c, 3)
        hp = lax.rem(c + 2, 3)
        base = wid * EW + c * CB2
        rb = h * CB2
        pltpu.make_async_copy(dsts_hbm.at[pl.ds(base, CB2)],
                              dsidx.at[h], sem_l).wait()
        pltpu.make_async_copy(score_hbm.at[pl.ds(base, CB2)],
                              scorebuf.at[h], sem_l).wait()
        pltpu.make_async_copy(comp_hbm.at[pl.ds(base, CB2)],
                              scatbuf.at[pl.ds(rb, CB2)], sem_l).wait()

        def _group(v, _):
            sl = pl.ds(v * 16, 16)
            dstvec = dsidx[h, sl]
            mvec = plsc.load_gather(m_v, [dstvec])
            ex = jnp.exp(scorebuf[h, sl] - mvec)
            plsc.addupdate_scatter(den_v, [dstvec], ex)
            for l in range(16):
                e = rb + v * 16 + l
                exl = ex[l]
                for j in range(D // 16):
                    slj = pl.ds(j * 16, 16)
                    scatbuf[e, slj] = scatbuf[e, slj] * exl
            return 0
        lax.fori_loop(0, CB2 // 16, _group, 0)

        @pl.when(c > 0)
        def _():
            pltpu.make_async_copy(
                scatbuf.at[pl.ds(hp * CB2, CB2)],
                acc.at[dsidx.at[hp]], sem_s).wait()

        pltpu.async_copy(scatbuf.at[pl.ds(rb, CB2)],
                         acc.at[dsidx.at[h]], sem_s, add=True)

        @pl.when(c + 2 < NCH2)
        def _():
            _load(c + 2)
        return 0
    lax.fori_loop(0, NCH2, _chunk, 0)

    hl = lax.rem(NCH2 - 1, 3)
    pltpu.make_async_copy(scatbuf.at[pl.ds(hl * CB2, CB2)],
                          acc.at[dsidx.at[hl]], sem_s).wait()

    plsc.subcore_barrier()
    row0 = sid * ROWS_PER_TILE
    pltpu.sync_copy(acc.at[pl.ds(row0, ROWS_PER_TILE)],
                    accn_hbm.at[cid].at[pl.ds(row0, ROWS_PER_TILE)])
    pltpu.sync_copy(den_v, den_hbm.at[cid].at[sid])


def _final_body(accn_ref, den_ref, w_ref, g_ref, b_ref, out_ref):
    num = (accn_ref[0] + accn_ref[1])[0:N]
    den = jnp.sum(den_ref[...], axis=(0, 1))[0:N, None]
    neigh = num / jnp.maximum(den, 1e-30)
    out = jnp.dot(neigh, w_ref[...], preferred_element_type=jnp.float32)
    mean = jnp.mean(out, axis=0, keepdims=True)
    var = jnp.mean((out - mean) ** 2, axis=0, keepdims=True)
    out = (out - mean) / jnp.sqrt(var + EPS) * g_ref[...] + b_ref[...]
    out_ref[...] = jnp.tanh(out)


def _final(accn, dens, neigh_w, bn_gamma, bn_beta):
    return pl.pallas_call(
        _final_body,
        out_shape=jax.ShapeDtypeStruct((N, D), jnp.float32),
    )(accn, dens, neigh_w, bn_gamma.reshape(1, D), bn_beta.reshape(1, D))


def kernel(ent_emb, rel_emb, edge_index, rel_id, neigh_w, bn_gamma, bn_beta):
    src = edge_index[0]
    dst = edge_index[1]
    pad = E_PAD - E
    zpad = jnp.zeros((pad,), jnp.int32)
    src_p = jnp.concatenate([src, zpad])
    dstg_p = jnp.concatenate([dst, zpad])
    dsts_p = jnp.concatenate([dst, jnp.full((pad,), TRASH, jnp.int32)])
    rel_p = jnp.concatenate([rel_id, zpad])

    src3 = src_p.reshape(NW, NCHUNK, CB)
    dstg3 = dstg_p.reshape(NW, NCHUNK, CB)
    score, comp, sums, cnts = _pass1(ent_emb, rel_emb, src3, dstg3, rel_p)
    m = _seg_mean(sums, cnts).reshape(N_PAD)
    accn, dens = _pass2(comp, dsts_p, score, m)
    return _final(accn, dens, neigh_w, bn_gamma, bn_beta)
